# baseline (speedup 1.0000x reference)
import jax
import jax.numpy as jnp
from jax import lax
from jax.experimental import pallas as pl
from jax.experimental.pallas import tpu as pltpu
from jax.experimental.pallas import tpu_sc as plsc

B, C, N = 4, 64, 65536
HID, OUT = 32, 8
TOPK = 1024
TILE = 8192

NCORE, NSUB, L = 2, 16, 16
BPC = B // NCORE
CHUNK = N // NSUB
CAND = 2 * TOPK
MYC = TOPK // NSUB

MININT = -(2**31)
MAXPOS = 0x7FFFFFFF


def _mlp_body(f_ref, w1_ref, b1_ref, w2_ref, b2_ref, *out_refs):
    f = f_ref[0]
    h = lax.dot_general(w1_ref[...], f, (((0,), (0,)), ((), ())),
                        preferred_element_type=jnp.float32)
    h = jnp.maximum(h + b1_ref[...], 0.0)
    t = lax.dot_general(w2_ref[...], h, (((0,), (0,)), ((), ())),
                        preferred_element_type=jnp.float32)
    t = t + b2_ref[...]
    for j in range(OUT):
        out_refs[j][...] = t[j]
    bits = lax.bitcast_convert_type(t[OUT - 1], jnp.int32)
    key = bits ^ (lax.shift_right_arithmetic(bits, 31) & jnp.int32(MAXPOS))
    out_refs[OUT][...] = key


def _mlp(features, W1, b1, W2, b2):
    flat = jax.ShapeDtypeStruct((B * N,), jnp.float32)
    return pl.pallas_call(
        _mlp_body,
        grid=(B, N // TILE),
        in_specs=[
            pl.BlockSpec((1, C, TILE), lambda b, n: (b, 0, n)),
            pl.BlockSpec((C, HID), lambda b, n: (0, 0)),
            pl.BlockSpec((HID, 1), lambda b, n: (0, 0)),
            pl.BlockSpec((HID, OUT), lambda b, n: (0, 0)),
            pl.BlockSpec((OUT, 1), lambda b, n: (0, 0)),
        ],
        out_specs=[
            pl.BlockSpec((TILE,), lambda b, n: (b * (N // TILE) + n,))
            for _ in range(OUT + 1)
        ],
        out_shape=[flat] * OUT + [jax.ShapeDtypeStruct((B * N,), jnp.int32)],
    )(features, W1, b1.reshape(HID, 1), W2, b2.reshape(OUT, 1))


def _thresh_body(k_ref, t_ref):
    for b in range(B):
        k = k_ref[pl.ds(b * N, N)]

        def bit_round(bit, tb):
            trial = tb | (jnp.int32(1) << (31 - bit))
            trial_s = trial ^ jnp.int32(MININT)
            cnt = jnp.sum((k >= trial_s).astype(jnp.int32))
            return lax.select(cnt >= TOPK, trial, tb)

        tb = lax.fori_loop(0, 32, bit_round, jnp.int32(0))
        t_ref[pl.ds(b * L, L)] = jnp.full((L,), tb ^ jnp.int32(MININT),
                                          jnp.int32)


def _thresh(keys):
    return pl.pallas_call(
        _thresh_body,
        grid=(1,),
        in_specs=[pl.BlockSpec((B * N,), lambda g: (0,))],
        out_specs=pl.BlockSpec((B * L,), lambda g: (0,)),
        out_shape=jax.ShapeDtypeStruct((B * L,), jnp.int32),
    )(keys)


def _sc_body(keys_hbm, thr_hbm, t0, t1, t2, t3, t4, t5, t6, t7, oflat_hbm,
             keys_v, gek_v, gei_v, ges_v,
             ck_v, cidx_v, rrow_v, tmp_v,
             counts_all_v, g1d_v, o2d_v, cols_v,
             counts_sh, candk_sh, candi_sh, sorted_sh, sem):
    ts = (t0, t1, t2, t3, t4, t5, t6, t7)
    c = lax.axis_index("c")
    s = lax.axis_index("s")
    iota = lax.iota(jnp.int32, L)

    for i in range(BPC):
        b = c * BPC + i
        base_n = b * N + s * CHUNK

        pltpu.sync_copy(keys_hbm.at[pl.ds(base_n, CHUNK)], keys_v)
        pltpu.sync_copy(thr_hbm.at[pl.ds(b * L, L)], tmp_v)
        t_splat = tmp_v[...]

        def compress(j, off_splat):
            k = keys_v[pl.ds(j * L, L)]
            idx = s * CHUNK + j * L + iota
            m = k >= t_splat
            pg = plsc.cumsum(m.astype(jnp.int32))
            tgt = off_splat + pg - 1
            plsc.store_scatter(gek_v, [tgt], k, mask=m)
            plsc.store_scatter(gei_v, [tgt], idx, mask=m)
            return off_splat + plsc.all_reduce_population_count(m)

        ge_splat = lax.fori_loop(0, CHUNK // L, compress,
                                 jnp.zeros((L,), jnp.int32))
        tmp_v[...] = ge_splat
        run_ge = tmp_v[...][0]

        def cnt_gt(q, acc):
            bk = gek_v[pl.ds(q * L, L)]
            valid = (q * L + iota) < ge_splat
            mg = (bk > t_splat) & valid
            return acc + plsc.all_reduce_population_count(mg)

        gt_splat = lax.fori_loop(0, (run_ge + L - 1) // L, cnt_gt,
                                 jnp.zeros((L,), jnp.int32))
        eq_splat = ge_splat - gt_splat

        tmp_v[...] = gt_splat
        pltpu.sync_copy(tmp_v, counts_sh.at[s])
        tmp_v[...] = eq_splat
        pltpu.sync_copy(tmp_v, counts_sh.at[NSUB + s])
        plsc.subcore_barrier()
        pltpu.sync_copy(counts_sh, counts_all_v)

        gt_base = jnp.int32(0)
        eq_base = jnp.int32(0)
        total_gt = jnp.int32(0)
        for v in range(NSUB):
            gcnt = counts_all_v[v][0]
            ecnt = counts_all_v[NSUB + v][0]
            before = (v < s).astype(jnp.int32)
            gt_base = gt_base + gcnt * before
            eq_base = eq_base + ecnt * before
            total_gt = total_gt + gcnt
        eq_base = eq_base + total_gt

        def fill(v, gtoff_splat):
            r = v // (128 // L)
            u = v % (128 // L)
            pos = v * L + iota
            bk = gek_v[pl.ds(v * L, L)]
            valid = pos < ge_splat
            mg = (bk > t_splat) & valid
            pgg = gtoff_splat + plsc.cumsum(mg.astype(jnp.int32))
            slot = jnp.where(mg, gt_base + pgg - 1, eq_base + pos - pgg)
            ok = valid & (slot < TOPK)
            slot = jnp.where(ok, slot, TOPK + (slot & (TOPK - 1)))
            ges_v[r, pl.ds(u * L, L)] = slot
            return gtoff_splat + plsc.all_reduce_population_count(mg)

        trips_fill = ((run_ge + 127) // 128) * (128 // L)
        lax.fori_loop(0, trips_fill, fill, jnp.zeros((L,), jnp.int32))

        def scat(r, _):
            pltpu.sync_copy(gek_v.at[pl.ds(r * 128, 128)],
                            candk_sh.at[ges_v.at[r]])
            pltpu.sync_copy(gei_v.at[pl.ds(r * 128, 128)],
                            candi_sh.at[ges_v.at[r]])
            return 0

        lax.fori_loop(0, (run_ge + 127) // 128, scat, 0)
        plsc.subcore_barrier()

        pltpu.sync_copy(candk_sh.at[pl.ds(0, TOPK)], ck_v)
        pltpu.sync_copy(candi_sh.at[pl.ds(s * MYC, MYC)], cidx_v)

        kis = [ck_v[pl.ds(s * MYC + t * L, L)] for t in range(MYC // L)]
        pos_is = [s * MYC + t * L + iota for t in range(MYC // L)]

        def jbody(j, ranks):
            jbase = j * L
            ranks = list(ranks)
            for r in range(L):
                gidx = jbase + ((iota + r) & (L - 1))
                kjr = plsc.load_gather(ck_v, [gidx])
                for t in range(MYC // L):
                    hit = (kjr > kis[t]) | ((kjr == kis[t])
                                            & (gidx < pos_is[t]))
                    ranks[t] = ranks[t] + hit.astype(jnp.int32)
            return tuple(ranks)

        ranks = lax.fori_loop(0, TOPK // L, jbody,
                              tuple(jnp.zeros((L,), jnp.int32)
                                    for _ in range(MYC // L)))
        for t in range(MYC // L):
            rrow_v[pl.ds(t * L, L)] = ranks[t]

        for v in range(MYC // L):
            idxv = cidx_v[pl.ds(v * L, L)]
            rnkv = rrow_v[pl.ds(v * L, L)]
            g1d_v[pl.ds(v * L, L)] = b * N + idxv
            for j in range(OUT):
                o2d_v[j, pl.ds(v * L, L)] = rnkv * OUT + j
        copies = [
            pltpu.async_copy(ts[j].at[g1d_v], cols_v.at[j], sem)
            for j in range(OUT)
        ]
        for cp in copies:
            cp.wait()
        for j in range(OUT):
            pltpu.sync_copy(cols_v.at[j], sorted_sh.at[o2d_v.at[j]])
        plsc.subcore_barrier()
        seg = OUT * TOPK // NSUB
        pltpu.sync_copy(
            sorted_sh.at[pl.ds(s * seg, seg)],
            oflat_hbm.at[pl.ds(b * OUT * TOPK + s * seg, seg)])
        plsc.subcore_barrier()


def _sc_select(keys_flat, thr_flat, t_chans):
    kern = pl.kernel(
        _sc_body,
        out_type=jax.ShapeDtypeStruct((B * TOPK * OUT,), jnp.float32),
        mesh=plsc.VectorSubcoreMesh(core_axis_name="c", subcore_axis_name="s",
                                    num_cores=NCORE, num_subcores=NSUB),
        compiler_params=pltpu.CompilerParams(needs_layout_passes=False),
        scratch_types=[
            pltpu.VMEM((CHUNK,), jnp.int32),
            pltpu.VMEM((CHUNK + L,), jnp.int32),
            pltpu.VMEM((CHUNK + L,), jnp.int32),
            pltpu.VMEM((CHUNK // 128, 128), jnp.int32),
            pltpu.VMEM((TOPK,), jnp.int32),
            pltpu.VMEM((MYC,), jnp.int32),
            pltpu.VMEM((MYC,), jnp.int32),
            pltpu.VMEM((L,), jnp.int32),
            pltpu.VMEM((2 * NSUB, L), jnp.int32),
            pltpu.VMEM((MYC,), jnp.int32),
            pltpu.VMEM((OUT, MYC), jnp.int32),
            pltpu.VMEM((OUT, MYC), jnp.float32),
            pltpu.VMEM_SHARED((2 * NSUB, L), jnp.int32),
            pltpu.VMEM_SHARED((CAND,), jnp.int32),
            pltpu.VMEM_SHARED((CAND,), jnp.int32),
            pltpu.VMEM_SHARED((OUT * TOPK,), jnp.float32),
            pltpu.SemaphoreType.DMA,
        ],
    )
    return kern(keys_flat, thr_flat, *t_chans)


def kernel(points, features, W1, b1, W2, b2):
    *t_chans, keys = _mlp(features, W1, b1, W2, b2)
    thr = _thresh(keys)
    out_flat = _sc_select(keys, thr, t_chans)
    return out_flat.reshape(B, TOPK, OUT)


# speedup vs baseline: 1.8554x; 1.8554x over previous
import jax
import jax.numpy as jnp
from jax import lax
from jax.experimental import pallas as pl
from jax.experimental.pallas import tpu as pltpu
from jax.experimental.pallas import tpu_sc as plsc

B, C, N = 4, 64, 65536
HID, OUT = 32, 8
TOPK = 1024
TILE = 8192

NCORE, NSUB, L = 2, 16, 16
BPC = B // NCORE
CHUNK = N // NSUB
CAND = 2 * TOPK
MYC = TOPK // NSUB

MININT = -(2**31)
MAXPOS = 0x7FFFFFFF


def _mlp_body(f_ref, w1_ref, b1_ref, w2_ref, b2_ref, *out_refs):
    f = f_ref[0]
    h = lax.dot_general(w1_ref[...], f, (((0,), (0,)), ((), ())),
                        preferred_element_type=jnp.float32)
    h = jnp.maximum(h + b1_ref[...], 0.0)
    t = lax.dot_general(w2_ref[...], h, (((0,), (0,)), ((), ())),
                        preferred_element_type=jnp.float32)
    t = t + b2_ref[...]
    for j in range(OUT):
        out_refs[j][...] = t[j]
    bits = lax.bitcast_convert_type(t[OUT - 1], jnp.int32)
    key = bits ^ (lax.shift_right_arithmetic(bits, 31) & jnp.int32(MAXPOS))
    out_refs[OUT][...] = key


def _mlp(features, W1, b1, W2, b2):
    flat = jax.ShapeDtypeStruct((B * N,), jnp.float32)
    return pl.pallas_call(
        _mlp_body,
        grid=(B, N // TILE),
        in_specs=[
            pl.BlockSpec((1, C, TILE), lambda b, n: (b, 0, n)),
            pl.BlockSpec((C, HID), lambda b, n: (0, 0)),
            pl.BlockSpec((HID, 1), lambda b, n: (0, 0)),
            pl.BlockSpec((HID, OUT), lambda b, n: (0, 0)),
            pl.BlockSpec((OUT, 1), lambda b, n: (0, 0)),
        ],
        out_specs=[
            pl.BlockSpec((TILE,), lambda b, n: (b * (N // TILE) + n,))
            for _ in range(OUT + 1)
        ],
        out_shape=[flat] * OUT + [jax.ShapeDtypeStruct((B * N,), jnp.int32)],
    )(features, W1, b1.reshape(HID, 1), W2, b2.reshape(OUT, 1))


def _thresh_body(k_ref, t_ref):
    for b in range(B):
        k = k_ref[pl.ds(b * N, N)]

        def bit_round(bit, tb):
            trial = tb | (jnp.int32(1) << (31 - bit))
            trial_s = trial ^ jnp.int32(MININT)
            cnt = jnp.sum((k >= trial_s).astype(jnp.int32))
            return lax.select(cnt >= TOPK, trial, tb)

        tb = lax.fori_loop(0, 32, bit_round, jnp.int32(0))
        t_ref[pl.ds(b * L, L)] = jnp.full((L,), tb ^ jnp.int32(MININT),
                                          jnp.int32)


def _thresh(keys):
    return pl.pallas_call(
        _thresh_body,
        grid=(1,),
        in_specs=[pl.BlockSpec((B * N,), lambda g: (0,))],
        out_specs=pl.BlockSpec((B * L,), lambda g: (0,)),
        out_shape=jax.ShapeDtypeStruct((B * L,), jnp.int32),
    )(keys)


def _sc_body(keys_hbm, thr_hbm, t0, t1, t2, t3, t4, t5, t6, t7, oflat_hbm,
             keys_v, gek_v, gei_v, ges_v,
             ck_v, cidx_v, rrow_v, tmp_v,
             counts_all_v, g1d_v, o2d_v, cols_v,
             counts_sh, candk_sh, candi_sh, sorted_sh, sem):
    ts = (t0, t1, t2, t3, t4, t5, t6, t7)
    c = lax.axis_index("c")
    s = lax.axis_index("s")
    iota = lax.iota(jnp.int32, L)

    for i in range(BPC):
        b = c * BPC + i
        base_n = b * N + s * CHUNK

        pltpu.sync_copy(keys_hbm.at[pl.ds(base_n, CHUNK)], keys_v)
        pltpu.sync_copy(thr_hbm.at[pl.ds(b * L, L)], tmp_v)
        t_splat = tmp_v[...]

        def compress(j, off_splat):
            k = keys_v[pl.ds(j * L, L)]
            idx = s * CHUNK + j * L + iota
            m = k >= t_splat
            pg = plsc.cumsum(m.astype(jnp.int32))
            tgt = off_splat + pg - 1
            plsc.store_scatter(gek_v, [tgt], k, mask=m)
            plsc.store_scatter(gei_v, [tgt], idx, mask=m)
            return off_splat + plsc.all_reduce_population_count(m)

        ge_splat = lax.fori_loop(0, CHUNK // L, compress,
                                 jnp.zeros((L,), jnp.int32))
        tmp_v[...] = ge_splat
        run_ge = tmp_v[...][0]

        def cnt_gt(q, acc):
            bk = gek_v[pl.ds(q * L, L)]
            valid = (q * L + iota) < ge_splat
            mg = (bk > t_splat) & valid
            return acc + plsc.all_reduce_population_count(mg)

        gt_splat = lax.fori_loop(0, (run_ge + L - 1) // L, cnt_gt,
                                 jnp.zeros((L,), jnp.int32))
        eq_splat = ge_splat - gt_splat

        tmp_v[...] = gt_splat
        pltpu.sync_copy(tmp_v, counts_sh.at[s])
        tmp_v[...] = eq_splat
        pltpu.sync_copy(tmp_v, counts_sh.at[NSUB + s])
        plsc.subcore_barrier()
        pltpu.sync_copy(counts_sh, counts_all_v)

        gt_base = jnp.int32(0)
        eq_base = jnp.int32(0)
        total_gt = jnp.int32(0)
        for v in range(NSUB):
            gcnt = counts_all_v[v][0]
            ecnt = counts_all_v[NSUB + v][0]
            before = (v < s).astype(jnp.int32)
            gt_base = gt_base + gcnt * before
            eq_base = eq_base + ecnt * before
            total_gt = total_gt + gcnt
        eq_base = eq_base + total_gt

        def fill(v, gtoff_splat):
            r = v // (128 // L)
            u = v % (128 // L)
            pos = v * L + iota
            bk = gek_v[pl.ds(v * L, L)]
            valid = pos < ge_splat
            mg = (bk > t_splat) & valid
            pgg = gtoff_splat + plsc.cumsum(mg.astype(jnp.int32))
            slot = jnp.where(mg, gt_base + pgg - 1, eq_base + pos - pgg)
            ok = valid & (slot < TOPK)
            slot = jnp.where(ok, slot, TOPK + (slot & (TOPK - 1)))
            ges_v[r, pl.ds(u * L, L)] = slot
            return gtoff_splat + plsc.all_reduce_population_count(mg)

        trips_fill = ((run_ge + 127) // 128) * (128 // L)
        lax.fori_loop(0, trips_fill, fill, jnp.zeros((L,), jnp.int32))

        def scat(r, _):
            pltpu.sync_copy(gek_v.at[pl.ds(r * 128, 128)],
                            candk_sh.at[ges_v.at[r]])
            pltpu.sync_copy(gei_v.at[pl.ds(r * 128, 128)],
                            candi_sh.at[ges_v.at[r]])
            return 0

        lax.fori_loop(0, (run_ge + 127) // 128, scat, 0)
        plsc.subcore_barrier()

        pltpu.sync_copy(candk_sh.at[pl.ds(0, TOPK)], ck_v)
        pltpu.sync_copy(candi_sh.at[pl.ds(s * MYC, MYC)], cidx_v)

        for i_blk in range(MYC // L):
            my0 = s * MYC + i_blk * L
            ki = ck_v[pl.ds(my0, L)]
            pos_i = my0 + iota

            def jbody(j, rank):
                jbase = j * L
                for r in range(L):
                    gidx = jbase + ((iota + r) & (L - 1))
                    kjr = plsc.load_gather(ck_v, [gidx])
                    hit = (kjr > ki) | ((kjr == ki) & (gidx < pos_i))
                    rank = rank + hit.astype(jnp.int32)
                return rank

            rank = lax.fori_loop(0, TOPK // L, jbody,
                                 jnp.zeros((L,), jnp.int32))
            rrow_v[pl.ds(i_blk * L, L)] = rank

        for v in range(MYC // L):
            idxv = cidx_v[pl.ds(v * L, L)]
            rnkv = rrow_v[pl.ds(v * L, L)]
            g1d_v[pl.ds(v * L, L)] = b * N + idxv
            for j in range(OUT):
                o2d_v[j, pl.ds(v * L, L)] = rnkv * OUT + j
        copies = [
            pltpu.async_copy(ts[j].at[g1d_v], cols_v.at[j], sem)
            for j in range(OUT)
        ]
        for cp in copies:
            cp.wait()
        for j in range(OUT):
            pltpu.sync_copy(cols_v.at[j], sorted_sh.at[o2d_v.at[j]])
        plsc.subcore_barrier()
        seg = OUT * TOPK // NSUB
        pltpu.sync_copy(
            sorted_sh.at[pl.ds(s * seg, seg)],
            oflat_hbm.at[pl.ds(b * OUT * TOPK + s * seg, seg)])
        plsc.subcore_barrier()


def _sc_select(keys_flat, thr_flat, t_chans):
    kern = pl.kernel(
        _sc_body,
        out_type=jax.ShapeDtypeStruct((B * TOPK * OUT,), jnp.float32),
        mesh=plsc.VectorSubcoreMesh(core_axis_name="c", subcore_axis_name="s",
                                    num_cores=NCORE, num_subcores=NSUB),
        compiler_params=pltpu.CompilerParams(needs_layout_passes=False),
        scratch_types=[
            pltpu.VMEM((CHUNK,), jnp.int32),
            pltpu.VMEM((CHUNK + L,), jnp.int32),
            pltpu.VMEM((CHUNK + L,), jnp.int32),
            pltpu.VMEM((CHUNK // 128, 128), jnp.int32),
            pltpu.VMEM((TOPK,), jnp.int32),
            pltpu.VMEM((MYC,), jnp.int32),
            pltpu.VMEM((MYC,), jnp.int32),
            pltpu.VMEM((L,), jnp.int32),
            pltpu.VMEM((2 * NSUB, L), jnp.int32),
            pltpu.VMEM((MYC,), jnp.int32),
            pltpu.VMEM((OUT, MYC), jnp.int32),
            pltpu.VMEM((OUT, MYC), jnp.float32),
            pltpu.VMEM_SHARED((2 * NSUB, L), jnp.int32),
            pltpu.VMEM_SHARED((CAND,), jnp.int32),
            pltpu.VMEM_SHARED((CAND,), jnp.int32),
            pltpu.VMEM_SHARED((OUT * TOPK,), jnp.float32),
            pltpu.SemaphoreType.DMA,
        ],
    )
    return kern(keys_flat, thr_flat, *t_chans)


def kernel(points, features, W1, b1, W2, b2):
    *t_chans, keys = _mlp(features, W1, b1, W2, b2)
    thr = _thresh(keys)
    out_flat = _sc_select(keys, thr, t_chans)
    return out_flat.reshape(B, TOPK, OUT)


# speedup vs baseline: 2.0204x; 1.0889x over previous
import jax
import jax.numpy as jnp
from jax import lax
from jax.experimental import pallas as pl
from jax.experimental.pallas import tpu as pltpu
from jax.experimental.pallas import tpu_sc as plsc

B, C, N = 4, 64, 65536
HID, OUT = 32, 8
TOPK = 1024
TILE = 8192

NCORE, NSUB, L = 2, 16, 16
BPC = B // NCORE
CHUNK = N // NSUB
CAND = 2 * TOPK
MYC = TOPK // NSUB

MININT = -(2**31)
MAXPOS = 0x7FFFFFFF


def _mlp_body(f_ref, w1_ref, b1_ref, w2_ref, b2_ref, *out_refs):
    f = f_ref[0]
    h = lax.dot_general(w1_ref[...], f, (((0,), (0,)), ((), ())),
                        preferred_element_type=jnp.float32)
    h = jnp.maximum(h + b1_ref[...], 0.0)
    t = lax.dot_general(w2_ref[...], h, (((0,), (0,)), ((), ())),
                        preferred_element_type=jnp.float32)
    t = t + b2_ref[...]
    for j in range(OUT):
        out_refs[j][...] = t[j]
    bits = lax.bitcast_convert_type(t[OUT - 1], jnp.int32)
    key = bits ^ (lax.shift_right_arithmetic(bits, 31) & jnp.int32(MAXPOS))
    out_refs[OUT][...] = key


def _mlp(features, W1, b1, W2, b2):
    flat = jax.ShapeDtypeStruct((B * N,), jnp.float32)
    return pl.pallas_call(
        _mlp_body,
        grid=(B, N // TILE),
        in_specs=[
            pl.BlockSpec((1, C, TILE), lambda b, n: (b, 0, n)),
            pl.BlockSpec((C, HID), lambda b, n: (0, 0)),
            pl.BlockSpec((HID, 1), lambda b, n: (0, 0)),
            pl.BlockSpec((HID, OUT), lambda b, n: (0, 0)),
            pl.BlockSpec((OUT, 1), lambda b, n: (0, 0)),
        ],
        out_specs=[
            pl.BlockSpec((TILE,), lambda b, n: (b * (N // TILE) + n,))
            for _ in range(OUT + 1)
        ],
        out_shape=[flat] * OUT + [jax.ShapeDtypeStruct((B * N,), jnp.int32)],
    )(features, W1, b1.reshape(HID, 1), W2, b2.reshape(OUT, 1))


def _thresh_body(k_ref, t_ref):
    ks = [k_ref[pl.ds(b * N, N)] for b in range(B)]

    def bit_round(bit, tbs):
        bitv = jnp.int32(1) << (31 - bit)
        out = []
        for b in range(B):
            trial = tbs[b] | bitv
            trial_s = trial ^ jnp.int32(MININT)
            cnt = jnp.sum((ks[b] >= trial_s).astype(jnp.int32))
            out.append(lax.select(cnt >= TOPK, trial, tbs[b]))
        return tuple(out)

    tbs = lax.fori_loop(0, 32, bit_round, (jnp.int32(0),) * B)
    for b in range(B):
        t_ref[pl.ds(b * L, L)] = jnp.full((L,), tbs[b] ^ jnp.int32(MININT),
                                          jnp.int32)


def _thresh(keys):
    return pl.pallas_call(
        _thresh_body,
        grid=(1,),
        in_specs=[pl.BlockSpec((B * N,), lambda g: (0,))],
        out_specs=pl.BlockSpec((B * L,), lambda g: (0,)),
        out_shape=jax.ShapeDtypeStruct((B * L,), jnp.int32),
    )(keys)


def _sc_body(keys_hbm, thr_hbm, t0, t1, t2, t3, t4, t5, t6, t7, oflat_hbm,
             keys_v, gek_v, gei_v, ges_v,
             ck_v, cidx_v, rrow_v, tmp_v,
             counts_all_v, g1d_v, o2d_v, cols_v,
             counts_sh, candk_sh, candi_sh, sorted_sh, sem):
    ts = (t0, t1, t2, t3, t4, t5, t6, t7)
    c = lax.axis_index("c")
    s = lax.axis_index("s")
    iota = lax.iota(jnp.int32, L)

    for i in range(BPC):
        b = c * BPC + i
        base_n = b * N + s * CHUNK

        pltpu.sync_copy(keys_hbm.at[pl.ds(base_n, CHUNK)], keys_v)
        pltpu.sync_copy(thr_hbm.at[pl.ds(b * L, L)], tmp_v)
        t_splat = tmp_v[...]

        def compress(j, off_splat):
            k = keys_v[pl.ds(j * L, L)]
            idx = s * CHUNK + j * L + iota
            m = k >= t_splat
            pg = plsc.cumsum(m.astype(jnp.int32))
            tgt = off_splat + pg - 1
            plsc.store_scatter(gek_v, [tgt], k, mask=m)
            plsc.store_scatter(gei_v, [tgt], idx, mask=m)
            return off_splat + plsc.all_reduce_population_count(m)

        ge_splat = lax.fori_loop(0, CHUNK // L, compress,
                                 jnp.zeros((L,), jnp.int32))
        tmp_v[...] = ge_splat
        run_ge = tmp_v[...][0]

        def cnt_gt(q, acc):
            bk = gek_v[pl.ds(q * L, L)]
            valid = (q * L + iota) < ge_splat
            mg = (bk > t_splat) & valid
            return acc + plsc.all_reduce_population_count(mg)

        gt_splat = lax.fori_loop(0, (run_ge + L - 1) // L, cnt_gt,
                                 jnp.zeros((L,), jnp.int32))
        eq_splat = ge_splat - gt_splat

        tmp_v[...] = gt_splat
        pltpu.sync_copy(tmp_v, counts_sh.at[s])
        tmp_v[...] = eq_splat
        pltpu.sync_copy(tmp_v, counts_sh.at[NSUB + s])
        plsc.subcore_barrier()
        pltpu.sync_copy(counts_sh, counts_all_v)

        gt_base = jnp.int32(0)
        eq_base = jnp.int32(0)
        total_gt = jnp.int32(0)
        for v in range(NSUB):
            gcnt = counts_all_v[v][0]
            ecnt = counts_all_v[NSUB + v][0]
            before = (v < s).astype(jnp.int32)
            gt_base = gt_base + gcnt * before
            eq_base = eq_base + ecnt * before
            total_gt = total_gt + gcnt
        eq_base = eq_base + total_gt

        def fill(v, gtoff_splat):
            r = v // (128 // L)
            u = v % (128 // L)
            pos = v * L + iota
            bk = gek_v[pl.ds(v * L, L)]
            valid = pos < ge_splat
            mg = (bk > t_splat) & valid
            pgg = gtoff_splat + plsc.cumsum(mg.astype(jnp.int32))
            slot = jnp.where(mg, gt_base + pgg - 1, eq_base + pos - pgg)
            ok = valid & (slot < TOPK)
            slot = jnp.where(ok, slot, TOPK + (slot & (TOPK - 1)))
            ges_v[r, pl.ds(u * L, L)] = slot
            return gtoff_splat + plsc.all_reduce_population_count(mg)

        trips_fill = ((run_ge + 127) // 128) * (128 // L)
        lax.fori_loop(0, trips_fill, fill, jnp.zeros((L,), jnp.int32))

        def scat(r, _):
            pltpu.sync_copy(gek_v.at[pl.ds(r * 128, 128)],
                            candk_sh.at[ges_v.at[r]])
            pltpu.sync_copy(gei_v.at[pl.ds(r * 128, 128)],
                            candi_sh.at[ges_v.at[r]])
            return 0

        lax.fori_loop(0, (run_ge + 127) // 128, scat, 0)
        plsc.subcore_barrier()

        pltpu.sync_copy(candk_sh.at[pl.ds(0, TOPK)], ck_v)
        pltpu.sync_copy(candi_sh.at[pl.ds(s * MYC, MYC)], cidx_v)

        for i_blk in range(MYC // L):
            my0 = s * MYC + i_blk * L
            ki = ck_v[pl.ds(my0, L)]
            pos_i = my0 + iota

            def jbody(j, rank):
                jbase = j * L
                for r in range(L):
                    gidx = jbase + ((iota + r) & (L - 1))
                    kjr = plsc.load_gather(ck_v, [gidx])
                    hit = (kjr > ki) | ((kjr == ki) & (gidx < pos_i))
                    rank = rank + hit.astype(jnp.int32)
                return rank

            rank = lax.fori_loop(0, TOPK // L, jbody,
                                 jnp.zeros((L,), jnp.int32))
            rrow_v[pl.ds(i_blk * L, L)] = rank

        for v in range(MYC // L):
            idxv = cidx_v[pl.ds(v * L, L)]
            rnkv = rrow_v[pl.ds(v * L, L)]
            g1d_v[pl.ds(v * L, L)] = b * N + idxv
            for j in range(OUT):
                o2d_v[j, pl.ds(v * L, L)] = rnkv * OUT + j
        copies = [
            pltpu.async_copy(ts[j].at[g1d_v], cols_v.at[j], sem)
            for j in range(OUT)
        ]
        for cp in copies:
            cp.wait()
        for j in range(OUT):
            pltpu.sync_copy(cols_v.at[j], sorted_sh.at[o2d_v.at[j]])
        plsc.subcore_barrier()
        seg = OUT * TOPK // NSUB
        pltpu.sync_copy(
            sorted_sh.at[pl.ds(s * seg, seg)],
            oflat_hbm.at[pl.ds(b * OUT * TOPK + s * seg, seg)])
        plsc.subcore_barrier()


def _sc_select(keys_flat, thr_flat, t_chans):
    kern = pl.kernel(
        _sc_body,
        out_type=jax.ShapeDtypeStruct((B * TOPK * OUT,), jnp.float32),
        mesh=plsc.VectorSubcoreMesh(core_axis_name="c", subcore_axis_name="s",
                                    num_cores=NCORE, num_subcores=NSUB),
        compiler_params=pltpu.CompilerParams(needs_layout_passes=False),
        scratch_types=[
            pltpu.VMEM((CHUNK,), jnp.int32),
            pltpu.VMEM((CHUNK + L,), jnp.int32),
            pltpu.VMEM((CHUNK + L,), jnp.int32),
            pltpu.VMEM((CHUNK // 128, 128), jnp.int32),
            pltpu.VMEM((TOPK,), jnp.int32),
            pltpu.VMEM((MYC,), jnp.int32),
            pltpu.VMEM((MYC,), jnp.int32),
            pltpu.VMEM((L,), jnp.int32),
            pltpu.VMEM((2 * NSUB, L), jnp.int32),
            pltpu.VMEM((MYC,), jnp.int32),
            pltpu.VMEM((OUT, MYC), jnp.int32),
            pltpu.VMEM((OUT, MYC), jnp.float32),
            pltpu.VMEM_SHARED((2 * NSUB, L), jnp.int32),
            pltpu.VMEM_SHARED((CAND,), jnp.int32),
            pltpu.VMEM_SHARED((CAND,), jnp.int32),
            pltpu.VMEM_SHARED((OUT * TOPK,), jnp.float32),
            pltpu.SemaphoreType.DMA,
        ],
    )
    return kern(keys_flat, thr_flat, *t_chans)


def kernel(points, features, W1, b1, W2, b2):
    *t_chans, keys = _mlp(features, W1, b1, W2, b2)
    thr = _thresh(keys)
    out_flat = _sc_select(keys, thr, t_chans)
    return out_flat.reshape(B, TOPK, OUT)


# speedup vs baseline: 2.5823x; 1.2781x over previous
import jax
import jax.numpy as jnp
from jax import lax
from jax.experimental import pallas as pl
from jax.experimental.pallas import tpu as pltpu
from jax.experimental.pallas import tpu_sc as plsc

B, C, N = 4, 64, 65536
HID, OUT = 32, 8
TOPK = 1024
TILE = 8192

NCORE, NSUB, L = 2, 16, 16
BPC = B // NCORE
CHUNK = N // NSUB
CAND = 2 * TOPK
MYC = TOPK // NSUB

MININT = -(2**31)
MAXPOS = 0x7FFFFFFF


def _mlp_body(f_ref, w1_ref, b1_ref, w2_ref, b2_ref, *out_refs):
    f = f_ref[0]
    h = lax.dot_general(w1_ref[...], f, (((0,), (0,)), ((), ())),
                        preferred_element_type=jnp.float32)
    h = jnp.maximum(h + b1_ref[...], 0.0)
    t = lax.dot_general(w2_ref[...], h, (((0,), (0,)), ((), ())),
                        preferred_element_type=jnp.float32)
    t = t + b2_ref[...]
    for j in range(OUT):
        out_refs[j][...] = t[j]
    bits = lax.bitcast_convert_type(t[OUT - 1], jnp.int32)
    key = bits ^ (lax.shift_right_arithmetic(bits, 31) & jnp.int32(MAXPOS))
    out_refs[OUT][...] = key


def _mlp(features, W1, b1, W2, b2):
    flat = jax.ShapeDtypeStruct((B * N,), jnp.float32)
    return pl.pallas_call(
        _mlp_body,
        grid=(B, N // TILE),
        in_specs=[
            pl.BlockSpec((1, C, TILE), lambda b, n: (b, 0, n)),
            pl.BlockSpec((C, HID), lambda b, n: (0, 0)),
            pl.BlockSpec((HID, 1), lambda b, n: (0, 0)),
            pl.BlockSpec((HID, OUT), lambda b, n: (0, 0)),
            pl.BlockSpec((OUT, 1), lambda b, n: (0, 0)),
        ],
        out_specs=[
            pl.BlockSpec((TILE,), lambda b, n: (b * (N // TILE) + n,))
            for _ in range(OUT + 1)
        ],
        out_shape=[flat] * OUT + [jax.ShapeDtypeStruct((B * N,), jnp.int32)],
    )(features, W1, b1.reshape(HID, 1), W2, b2.reshape(OUT, 1))


def _thresh_body(k_ref, t_ref):
    NR = N // 128
    ks = [k_ref[pl.ds(b * NR, NR), :] for b in range(B)]

    def bit_round(bit, tbs):
        bitv = jnp.int32(1) << (31 - bit)
        out = []
        for b in range(B):
            trial = tbs[b] | bitv
            trial_s = trial ^ jnp.int32(MININT)
            cnt = jnp.sum((ks[b] >= trial_s).astype(jnp.int32))
            out.append(lax.select(cnt >= TOPK, trial, tbs[b]))
        return tuple(out)

    tbs = lax.fori_loop(0, 32, bit_round, (jnp.int32(0),) * B)
    for b in range(B):
        t_ref[pl.ds(b * L, L)] = jnp.full((L,), tbs[b] ^ jnp.int32(MININT),
                                          jnp.int32)


def _thresh(keys):
    return pl.pallas_call(
        _thresh_body,
        grid=(1,),
        in_specs=[pl.BlockSpec((B * N // 128, 128), lambda g: (0, 0))],
        out_specs=pl.BlockSpec((B * L,), lambda g: (0,)),
        out_shape=jax.ShapeDtypeStruct((B * L,), jnp.int32),
    )(keys.reshape(B * N // 128, 128))


def _sc_body(keys_hbm, thr_hbm, t0, t1, t2, t3, t4, t5, t6, t7, oflat_hbm,
             keys_v, gek_v, gei_v, ges_v,
             ck_v, cidx_v, rrow_v, tmp_v,
             counts_all_v, g1d_v, o2d_v, cols_v,
             counts_sh, candk_sh, candi_sh, sorted_sh, sem):
    ts = (t0, t1, t2, t3, t4, t5, t6, t7)
    c = lax.axis_index("c")
    s = lax.axis_index("s")
    iota = lax.iota(jnp.int32, L)

    for i in range(BPC):
        b = c * BPC + i
        base_n = b * N + s * CHUNK

        pltpu.sync_copy(keys_hbm.at[pl.ds(base_n, CHUNK)], keys_v)
        pltpu.sync_copy(thr_hbm.at[pl.ds(b * L, L)], tmp_v)
        t_splat = tmp_v[...]

        def compress(j, off_splat):
            k = keys_v[pl.ds(j * L, L)]
            idx = s * CHUNK + j * L + iota
            m = k >= t_splat
            pg = plsc.cumsum(m.astype(jnp.int32))
            tgt = off_splat + pg - 1
            plsc.store_scatter(gek_v, [tgt], k, mask=m)
            plsc.store_scatter(gei_v, [tgt], idx, mask=m)
            return off_splat + plsc.all_reduce_population_count(m)

        ge_splat = lax.fori_loop(0, CHUNK // L, compress,
                                 jnp.zeros((L,), jnp.int32))
        tmp_v[...] = ge_splat
        run_ge = tmp_v[...][0]

        def cnt_gt(q, acc):
            bk = gek_v[pl.ds(q * L, L)]
            valid = (q * L + iota) < ge_splat
            mg = (bk > t_splat) & valid
            return acc + plsc.all_reduce_population_count(mg)

        gt_splat = lax.fori_loop(0, (run_ge + L - 1) // L, cnt_gt,
                                 jnp.zeros((L,), jnp.int32))
        eq_splat = ge_splat - gt_splat

        tmp_v[...] = gt_splat
        pltpu.sync_copy(tmp_v, counts_sh.at[s])
        tmp_v[...] = eq_splat
        pltpu.sync_copy(tmp_v, counts_sh.at[NSUB + s])
        plsc.subcore_barrier()
        pltpu.sync_copy(counts_sh, counts_all_v)

        gt_base = jnp.int32(0)
        eq_base = jnp.int32(0)
        total_gt = jnp.int32(0)
        for v in range(NSUB):
            gcnt = counts_all_v[v][0]
            ecnt = counts_all_v[NSUB + v][0]
            before = (v < s).astype(jnp.int32)
            gt_base = gt_base + gcnt * before
            eq_base = eq_base + ecnt * before
            total_gt = total_gt + gcnt
        eq_base = eq_base + total_gt

        def fill(v, gtoff_splat):
            r = v // (128 // L)
            u = v % (128 // L)
            pos = v * L + iota
            bk = gek_v[pl.ds(v * L, L)]
            valid = pos < ge_splat
            mg = (bk > t_splat) & valid
            pgg = gtoff_splat + plsc.cumsum(mg.astype(jnp.int32))
            slot = jnp.where(mg, gt_base + pgg - 1, eq_base + pos - pgg)
            ok = valid & (slot < TOPK)
            slot = jnp.where(ok, slot, TOPK + (slot & (TOPK - 1)))
            ges_v[r, pl.ds(u * L, L)] = slot
            return gtoff_splat + plsc.all_reduce_population_count(mg)

        trips_fill = ((run_ge + 127) // 128) * (128 // L)
        lax.fori_loop(0, trips_fill, fill, jnp.zeros((L,), jnp.int32))

        def scat(r, _):
            pltpu.sync_copy(gek_v.at[pl.ds(r * 128, 128)],
                            candk_sh.at[ges_v.at[r]])
            pltpu.sync_copy(gei_v.at[pl.ds(r * 128, 128)],
                            candi_sh.at[ges_v.at[r]])
            return 0

        lax.fori_loop(0, (run_ge + 127) // 128, scat, 0)
        plsc.subcore_barrier()

        pltpu.sync_copy(candk_sh.at[pl.ds(0, TOPK)], ck_v)
        pltpu.sync_copy(candi_sh.at[pl.ds(s * MYC, MYC)], cidx_v)

        for i_blk in range(MYC // L):
            my0 = s * MYC + i_blk * L
            ki = ck_v[pl.ds(my0, L)]
            pos_i = my0 + iota

            def jbody(j, rank):
                jbase = j * L
                for r in range(L):
                    gidx = jbase + ((iota + r) & (L - 1))
                    kjr = plsc.load_gather(ck_v, [gidx])
                    hit = (kjr > ki) | ((kjr == ki) & (gidx < pos_i))
                    rank = rank + hit.astype(jnp.int32)
                return rank

            rank = lax.fori_loop(0, TOPK // L, jbody,
                                 jnp.zeros((L,), jnp.int32))
            rrow_v[pl.ds(i_blk * L, L)] = rank

        for v in range(MYC // L):
            idxv = cidx_v[pl.ds(v * L, L)]
            rnkv = rrow_v[pl.ds(v * L, L)]
            g1d_v[pl.ds(v * L, L)] = b * N + idxv
            for j in range(OUT):
                o2d_v[j, pl.ds(v * L, L)] = rnkv * OUT + j
        copies = [
            pltpu.async_copy(ts[j].at[g1d_v], cols_v.at[j], sem)
            for j in range(OUT)
        ]
        for cp in copies:
            cp.wait()
        for j in range(OUT):
            pltpu.sync_copy(cols_v.at[j], sorted_sh.at[o2d_v.at[j]])
        plsc.subcore_barrier()
        seg = OUT * TOPK // NSUB
        pltpu.sync_copy(
            sorted_sh.at[pl.ds(s * seg, seg)],
            oflat_hbm.at[pl.ds(b * OUT * TOPK + s * seg, seg)])
        plsc.subcore_barrier()


def _sc_select(keys_flat, thr_flat, t_chans):
    kern = pl.kernel(
        _sc_body,
        out_type=jax.ShapeDtypeStruct((B * TOPK * OUT,), jnp.float32),
        mesh=plsc.VectorSubcoreMesh(core_axis_name="c", subcore_axis_name="s",
                                    num_cores=NCORE, num_subcores=NSUB),
        compiler_params=pltpu.CompilerParams(needs_layout_passes=False),
        scratch_types=[
            pltpu.VMEM((CHUNK,), jnp.int32),
            pltpu.VMEM((CHUNK + L,), jnp.int32),
            pltpu.VMEM((CHUNK + L,), jnp.int32),
            pltpu.VMEM((CHUNK // 128, 128), jnp.int32),
            pltpu.VMEM((TOPK,), jnp.int32),
            pltpu.VMEM((MYC,), jnp.int32),
            pltpu.VMEM((MYC,), jnp.int32),
            pltpu.VMEM((L,), jnp.int32),
            pltpu.VMEM((2 * NSUB, L), jnp.int32),
            pltpu.VMEM((MYC,), jnp.int32),
            pltpu.VMEM((OUT, MYC), jnp.int32),
            pltpu.VMEM((OUT, MYC), jnp.float32),
            pltpu.VMEM_SHARED((2 * NSUB, L), jnp.int32),
            pltpu.VMEM_SHARED((CAND,), jnp.int32),
            pltpu.VMEM_SHARED((CAND,), jnp.int32),
            pltpu.VMEM_SHARED((OUT * TOPK,), jnp.float32),
            pltpu.SemaphoreType.DMA,
        ],
    )
    return kern(keys_flat, thr_flat, *t_chans)


def kernel(points, features, W1, b1, W2, b2):
    *t_chans, keys = _mlp(features, W1, b1, W2, b2)
    thr = _thresh(keys)
    out_flat = _sc_select(keys, thr, t_chans)
    return out_flat.reshape(B, TOPK, OUT)


# speedup vs baseline: 2.6392x; 1.0221x over previous
import jax
import jax.numpy as jnp
from jax import lax
from jax.experimental import pallas as pl
from jax.experimental.pallas import tpu as pltpu
from jax.experimental.pallas import tpu_sc as plsc

B, C, N = 4, 64, 65536
HID, OUT = 32, 8
TOPK = 1024
TILE = 8192

NCORE, NSUB, L = 2, 16, 16
BPC = B // NCORE
CHUNK = N // NSUB
CAND = 2 * TOPK
MYC = TOPK // NSUB

MININT = -(2**31)
MAXPOS = 0x7FFFFFFF


def _mlp_body(f_ref, w1_ref, b1_ref, w2_ref, b2_ref, *out_refs):
    f = f_ref[0]
    h = lax.dot_general(w1_ref[...], f, (((0,), (0,)), ((), ())),
                        preferred_element_type=jnp.float32)
    h = jnp.maximum(h + b1_ref[...], 0.0)
    t = lax.dot_general(w2_ref[...], h, (((0,), (0,)), ((), ())),
                        preferred_element_type=jnp.float32)
    t = t + b2_ref[...]
    for j in range(OUT):
        out_refs[j][...] = t[j]
    bits = lax.bitcast_convert_type(t[OUT - 1], jnp.int32)
    key = bits ^ (lax.shift_right_arithmetic(bits, 31) & jnp.int32(MAXPOS))
    out_refs[OUT][...] = key


def _mlp(features, W1, b1, W2, b2):
    flat = jax.ShapeDtypeStruct((B * N,), jnp.float32)
    return pl.pallas_call(
        _mlp_body,
        grid=(B, N // TILE),
        in_specs=[
            pl.BlockSpec((1, C, TILE), lambda b, n: (b, 0, n)),
            pl.BlockSpec((C, HID), lambda b, n: (0, 0)),
            pl.BlockSpec((HID, 1), lambda b, n: (0, 0)),
            pl.BlockSpec((HID, OUT), lambda b, n: (0, 0)),
            pl.BlockSpec((OUT, 1), lambda b, n: (0, 0)),
        ],
        out_specs=[
            pl.BlockSpec((TILE,), lambda b, n: (b * (N // TILE) + n,))
            for _ in range(OUT + 1)
        ],
        out_shape=[flat] * OUT + [jax.ShapeDtypeStruct((B * N,), jnp.int32)],
    )(features, W1, b1.reshape(HID, 1), W2, b2.reshape(OUT, 1))


def _thresh_body(k_ref, t_ref):
    NR = N // 128
    ks = [k_ref[pl.ds(b * NR, NR), :] for b in range(B)]

    def bit_round(bit, tbs):
        bitv = jnp.int32(1) << (31 - bit)
        out = []
        for b in range(B):
            trial = tbs[b] | bitv
            trial_s = trial ^ jnp.int32(MININT)
            cnt = jnp.sum((ks[b] >= trial_s).astype(jnp.int32))
            out.append(lax.select(cnt >= TOPK, trial, tbs[b]))
        return tuple(out)

    tbs = lax.fori_loop(0, 32, bit_round, (jnp.int32(0),) * B)
    for b in range(B):
        t_ref[pl.ds(b * L, L)] = jnp.full((L,), tbs[b] ^ jnp.int32(MININT),
                                          jnp.int32)


def _thresh(keys):
    return pl.pallas_call(
        _thresh_body,
        grid=(1,),
        in_specs=[pl.BlockSpec((B * N // 128, 128), lambda g: (0, 0))],
        out_specs=pl.BlockSpec((B * L,), lambda g: (0,)),
        out_shape=jax.ShapeDtypeStruct((B * L,), jnp.int32),
    )(keys.reshape(B * N // 128, 128))


def _sc_body(keys_hbm, thr_hbm, t0, t1, t2, t3, t4, t5, t6, t7, oflat_hbm,
             keys_v, gek_v, gei_v, ges_v,
             ck_v, cidx_v, tmp_v,
             counts_all_v, g1d_v, o2d_v, cols_v,
             counts_sh, candk_sh, candi_sh, sorted_sh, sem):
    ts = (t0, t1, t2, t3, t4, t5, t6, t7)
    c = lax.axis_index("c")
    s = lax.axis_index("s")
    iota = lax.iota(jnp.int32, L)

    for i in range(BPC):
        b = c * BPC + i
        base_n = b * N + s * CHUNK

        pltpu.sync_copy(keys_hbm.at[pl.ds(base_n, CHUNK)], keys_v)
        pltpu.sync_copy(thr_hbm.at[pl.ds(b * L, L)], tmp_v)
        t_splat = tmp_v[...]

        def compress(j, off_splat):
            k = keys_v[pl.ds(j * L, L)]
            idx = s * CHUNK + j * L + iota
            m = k >= t_splat
            pg = plsc.cumsum(m.astype(jnp.int32))
            tgt = off_splat + pg - 1
            plsc.store_scatter(gek_v, [tgt], k, mask=m)
            plsc.store_scatter(gei_v, [tgt], idx, mask=m)
            return off_splat + plsc.all_reduce_population_count(m)

        ge_splat = lax.fori_loop(0, CHUNK // L, compress,
                                 jnp.zeros((L,), jnp.int32))
        tmp_v[...] = ge_splat
        run_ge = tmp_v[...][0]

        def cnt_gt(q, acc):
            bk = gek_v[pl.ds(q * L, L)]
            valid = (q * L + iota) < ge_splat
            mg = (bk > t_splat) & valid
            return acc + plsc.all_reduce_population_count(mg)

        gt_splat = lax.fori_loop(0, (run_ge + L - 1) // L, cnt_gt,
                                 jnp.zeros((L,), jnp.int32))
        eq_splat = ge_splat - gt_splat

        tmp_v[...] = gt_splat
        pltpu.sync_copy(tmp_v, counts_sh.at[s])
        tmp_v[...] = eq_splat
        pltpu.sync_copy(tmp_v, counts_sh.at[NSUB + s])
        plsc.subcore_barrier()
        pltpu.sync_copy(counts_sh, counts_all_v)

        gt_base = jnp.int32(0)
        eq_base = jnp.int32(0)
        total_gt = jnp.int32(0)
        for v in range(NSUB):
            gcnt = counts_all_v[v][0]
            ecnt = counts_all_v[NSUB + v][0]
            before = (v < s).astype(jnp.int32)
            gt_base = gt_base + gcnt * before
            eq_base = eq_base + ecnt * before
            total_gt = total_gt + gcnt
        eq_base = eq_base + total_gt

        def fill(v, gtoff_splat):
            r = v // (128 // L)
            u = v % (128 // L)
            pos = v * L + iota
            bk = gek_v[pl.ds(v * L, L)]
            valid = pos < ge_splat
            mg = (bk > t_splat) & valid
            pgg = gtoff_splat + plsc.cumsum(mg.astype(jnp.int32))
            slot = jnp.where(mg, gt_base + pgg - 1, eq_base + pos - pgg)
            ok = valid & (slot < TOPK)
            slot = jnp.where(ok, slot, TOPK + (slot & (TOPK - 1)))
            ges_v[r, pl.ds(u * L, L)] = slot
            return gtoff_splat + plsc.all_reduce_population_count(mg)

        trips_fill = ((run_ge + 127) // 128) * (128 // L)
        lax.fori_loop(0, trips_fill, fill, jnp.zeros((L,), jnp.int32))

        def scat(r, _):
            pltpu.sync_copy(gek_v.at[pl.ds(r * 128, 128)],
                            candk_sh.at[ges_v.at[r]])
            pltpu.sync_copy(gei_v.at[pl.ds(r * 128, 128)],
                            candi_sh.at[ges_v.at[r]])
            return 0

        lax.fori_loop(0, (run_ge + 127) // 128, scat, 0)
        plsc.subcore_barrier()

        pltpu.sync_copy(candk_sh.at[pl.ds(0, TOPK)], ck_v)
        pltpu.sync_copy(candi_sh.at[pl.ds(s * MYC, MYC)], cidx_v)
        for v in range(MYC // L):
            idxv = cidx_v[pl.ds(v * L, L)]
            g1d_v[pl.ds(v * L, L)] = b * N + idxv
        copies = [
            pltpu.async_copy(ts[j].at[g1d_v], cols_v.at[j], sem)
            for j in range(OUT)
        ]

        for i_blk in range(MYC // L):
            my0 = s * MYC + i_blk * L
            ki = ck_v[pl.ds(my0, L)]
            pos_i = my0 + iota

            def jbody(j, rank):
                jbase = j * L
                for r in range(L):
                    gidx = jbase + ((iota + r) & (L - 1))
                    kjr = plsc.load_gather(ck_v, [gidx])
                    hit = (kjr > ki) | ((kjr == ki) & (gidx < pos_i))
                    rank = rank + hit.astype(jnp.int32)
                return rank

            rank = lax.fori_loop(0, TOPK // L, jbody,
                                 jnp.zeros((L,), jnp.int32))
            for j in range(OUT):
                o2d_v[j, pl.ds(i_blk * L, L)] = rank * OUT + j

        for cp in copies:
            cp.wait()
        for j in range(OUT):
            pltpu.sync_copy(cols_v.at[j], sorted_sh.at[o2d_v.at[j]])
        plsc.subcore_barrier()
        seg = OUT * TOPK // NSUB
        pltpu.sync_copy(
            sorted_sh.at[pl.ds(s * seg, seg)],
            oflat_hbm.at[pl.ds(b * OUT * TOPK + s * seg, seg)])
        plsc.subcore_barrier()


def _sc_select(keys_flat, thr_flat, t_chans):
    kern = pl.kernel(
        _sc_body,
        out_type=jax.ShapeDtypeStruct((B * TOPK * OUT,), jnp.float32),
        mesh=plsc.VectorSubcoreMesh(core_axis_name="c", subcore_axis_name="s",
                                    num_cores=NCORE, num_subcores=NSUB),
        compiler_params=pltpu.CompilerParams(needs_layout_passes=False),
        scratch_types=[
            pltpu.VMEM((CHUNK,), jnp.int32),
            pltpu.VMEM((CHUNK + L,), jnp.int32),
            pltpu.VMEM((CHUNK + L,), jnp.int32),
            pltpu.VMEM((CHUNK // 128, 128), jnp.int32),
            pltpu.VMEM((TOPK,), jnp.int32),
            pltpu.VMEM((MYC,), jnp.int32),
            pltpu.VMEM((L,), jnp.int32),
            pltpu.VMEM((2 * NSUB, L), jnp.int32),
            pltpu.VMEM((MYC,), jnp.int32),
            pltpu.VMEM((OUT, MYC), jnp.int32),
            pltpu.VMEM((OUT, MYC), jnp.float32),
            pltpu.VMEM_SHARED((2 * NSUB, L), jnp.int32),
            pltpu.VMEM_SHARED((CAND,), jnp.int32),
            pltpu.VMEM_SHARED((CAND,), jnp.int32),
            pltpu.VMEM_SHARED((OUT * TOPK,), jnp.float32),
            pltpu.SemaphoreType.DMA,
        ],
    )
    return kern(keys_flat, thr_flat, *t_chans)


def kernel(points, features, W1, b1, W2, b2):
    *t_chans, keys = _mlp(features, W1, b1, W2, b2)
    thr = _thresh(keys)
    out_flat = _sc_select(keys, thr, t_chans)
    return out_flat.reshape(B, TOPK, OUT)
